# baseline (speedup 1.0000x reference)
import functools

import jax
import jax.numpy as jnp
import numpy as np
from jax.experimental import pallas as pl

B = 16
K = 16
H = 128
NGDF = 16
EPS = 1e-5


_SIG_D = np.float32(8.0 / NGDF)
_SIG_A = np.float32(2.0 / NGDF)


def _mu_row(d_min, d_max):
    step = (d_max - d_min) / (NGDF - 1)
    i = jax.lax.broadcasted_iota(jnp.int32, (1, NGDF), 1).astype(jnp.float32)
    return d_min + i * step


def _edge_h1(g1, g2, vn, w_ref, b1_ref, vt, et):
    vn_e = jnp.broadcast_to(vn[:, None, :], (vt, K, 8)).reshape(et, 8)
    dvec = g2[:, 0:3] - vn_e[:, 0:3]
    d2 = jnp.sum(dvec * dvec, axis=1, keepdims=True)
    dist = jnp.sqrt(d2)
    ang = jnp.sum(dvec * vn_e[:, 3:6], axis=1, keepdims=True) / dist
    mu_d = _mu_row(0.0, 8.0)
    mu_a = _mu_row(-1.0, 1.0)
    enc_d = jnp.exp(-(((dist - mu_d) / _SIG_D) ** 2))
    enc_a = jnp.exp(-(((ang - mu_a) / _SIG_A) ** 2))
    enc = jnp.concatenate([enc_d, enc_a], axis=1)
    return g1 + jnp.dot(enc, w_ref[...], preferred_element_type=jnp.float32) + b1_ref[...]


def _e1_body(g1_ref, g2_ref, vn_ref, w_ref, b1_ref, out_ref, *, vt, et):
    i = pl.program_id(0)

    @pl.when(i == 0)
    def _():
        out_ref[...] = jnp.zeros_like(out_ref)

    h1 = _edge_h1(g1_ref[...], g2_ref[...], vn_ref[...], w_ref, b1_ref, vt, et)
    out_ref[0:1, :] += jnp.sum(h1, axis=0, keepdims=True)
    out_ref[1:2, :] += jnp.sum(h1 * h1, axis=0, keepdims=True)


def _e2_body(g1_ref, g2_ref, vn_ref, w_ref, b1_ref, a1_ref, c1_ref,
             w2_ref, b2_ref, out_ref, *, vt, et):
    i = pl.program_id(0)

    @pl.when(i == 0)
    def _():
        out_ref[...] = jnp.zeros_like(out_ref)

    h1 = _edge_h1(g1_ref[...], g2_ref[...], vn_ref[...], w_ref, b1_ref, vt, et)
    h = h1 * a1_ref[...] + c1_ref[...]
    h = h * jax.nn.sigmoid(h)
    h2 = jnp.dot(h, w2_ref[...], preferred_element_type=jnp.float32) + b2_ref[...]
    out_ref[0:1, :] += jnp.sum(h2, axis=0, keepdims=True)
    out_ref[1:2, :] += jnp.sum(h2 * h2, axis=0, keepdims=True)


def _e3_body(g1_ref, g2_ref, vn_ref, w_ref, b1_ref, a1_ref, c1_ref,
             w2_ref, b2_ref, a2_ref, c2_ref, out_ref, *, vt, et):
    h1 = _edge_h1(g1_ref[...], g2_ref[...], vn_ref[...], w_ref, b1_ref, vt, et)
    h = h1 * a1_ref[...] + c1_ref[...]
    h = h * jax.nn.sigmoid(h)
    h2 = jnp.dot(h, w2_ref[...], preferred_element_type=jnp.float32) + b2_ref[...]
    y = h2 * a2_ref[...] + c2_ref[...]
    f2 = y[:, :H]
    c2 = y[:, H:]
    glu = jax.nn.sigmoid(f2) * jax.nn.softplus(c2)
    out_ref[...] = jnp.sum(glu.reshape(vt, K, H), axis=1)


def _bn_coefs(s, sq, count, g, be):
    mean = s / count
    var = sq / count - mean * mean
    a = g / jnp.sqrt(var + EPS)
    c = be - mean * a
    return a[None, :], c[None, :]


def _pick_vt(n_verts):
    for vt in (400, 200, 80, 40, 8):
        if n_verts % vt == 0:
            return vt
    return 8


def _edge_stage(z, g2, vn, sw1r, sb1, sg1, sbe1, sw2, sb2, sg2, sbe2, idxf):
    n_verts = vn.shape[0]
    e = idxf.shape[0]
    vt = _pick_vt(n_verts)
    et = vt * K
    grid = n_verts // vt

    g1 = z[idxf]

    espec = [
        pl.BlockSpec((et, H), lambda i: (i, 0)),
        pl.BlockSpec((et, 16), lambda i: (i, 0)),
        pl.BlockSpec((vt, 8), lambda i: (i, 0)),
        pl.BlockSpec((2 * NGDF, H), lambda i: (0, 0)),
        pl.BlockSpec((1, H), lambda i: (0, 0)),
    ]
    acc1 = pl.pallas_call(
        functools.partial(_e1_body, vt=vt, et=et),
        grid=(grid,),
        in_specs=espec,
        out_specs=pl.BlockSpec((2, H), lambda i: (0, 0)),
        out_shape=jax.ShapeDtypeStruct((2, H), jnp.float32),
    )(g1, g2, vn, sw1r, sb1[None, :])
    a1, c1 = _bn_coefs(acc1[0], acc1[1], e, sg1, sbe1)

    espec2 = espec + [
        pl.BlockSpec((1, H), lambda i: (0, 0)),
        pl.BlockSpec((1, H), lambda i: (0, 0)),
        pl.BlockSpec((H, 2 * H), lambda i: (0, 0)),
        pl.BlockSpec((1, 2 * H), lambda i: (0, 0)),
    ]
    acc2 = pl.pallas_call(
        functools.partial(_e2_body, vt=vt, et=et),
        grid=(grid,),
        in_specs=espec2,
        out_specs=pl.BlockSpec((2, 2 * H), lambda i: (0, 0)),
        out_shape=jax.ShapeDtypeStruct((2, 2 * H), jnp.float32),
    )(g1, g2, vn, sw1r, sb1[None, :], a1, c1, sw2, sb2[None, :])
    a2, c2 = _bn_coefs(acc2[0], acc2[1], e, sg2, sbe2)

    espec3 = espec2 + [
        pl.BlockSpec((1, 2 * H), lambda i: (0, 0)),
        pl.BlockSpec((1, 2 * H), lambda i: (0, 0)),
    ]
    hcg = pl.pallas_call(
        functools.partial(_e3_body, vt=vt, et=et),
        grid=(grid,),
        in_specs=espec3,
        out_specs=pl.BlockSpec((vt, H), lambda i: (i, 0)),
        out_shape=jax.ShapeDtypeStruct((n_verts, H), jnp.float32),
    )(g1, g2, vn, sw1r, sb1[None, :], a1, c1, sw2, sb2[None, :], a2, c2)
    return hcg


def _bn(x, g, b):
    m = jnp.mean(x, axis=0)
    v = jnp.var(x, axis=0)
    return g * (x - m) / jnp.sqrt(v + EPS) + b


def _mlp(x, w1, b1, g1, be1, w2, b2, g2, be2):
    h = x @ w1 + b1
    h = _bn(h, g1, be1)
    h = jax.nn.silu(h)
    h = h @ w2 + b2
    h = _bn(h, g2, be2)
    return h


def kernel(graph_x, surface_x, verts, node_pos, vnormals,
           cw1, cb1, cg1, cbe1, cw2, cb2, cg2, cbe2,
           gw1, gb1, gg1, gbe1, gw2, gb2, gg2, gbe2,
           sw1, sb1, sg1, sbe1, sw2, sb2, sg2, sbe2,
           fw1, fb1, fg1, fbe1, fw2, fb2, fg2, fbe2):
    n_total = verts.shape[0]
    n = n_total // B

    h_geom0 = _mlp(surface_x, gw1, gb1, gg1, gbe1, gw2, gb2, gg2, gbe2)
    h_chem2 = _mlp(graph_x, cw1, cb1, cg1, cbe1, cw2, cb2, cg2, cbe2)
    f, c = jnp.split(h_chem2, 2, axis=-1)
    h_chem = jax.nn.sigmoid(f) * jax.nn.softplus(c)

    v4 = verts.reshape(B, n, 3)
    p4 = node_pos.reshape(B, n, 3)
    d = jnp.sqrt(jnp.sum((v4[:, :, None, :] - p4[:, None, :, :]) ** 2, axis=-1))
    idx = jax.lax.top_k(-d, K)[1]
    idxf = (idx + (jnp.arange(B, dtype=jnp.int32) * n)[:, None, None]).reshape(-1)
    idxf = idxf.astype(jnp.int32)

    z = h_chem @ sw1[:H]
    e = idxf.shape[0]
    g2 = jnp.concatenate(
        [node_pos[idxf], jnp.zeros((e, 13), jnp.float32)], axis=1)
    vn = jnp.concatenate(
        [verts, vnormals, jnp.zeros((n_total, 2), jnp.float32)], axis=1)

    hcg = _edge_stage(z, g2, vn, sw1[H:], sb1, sg1, sbe1,
                      sw2, sb2, sg2, sbe2, idxf)

    h_geom = _mlp(jnp.concatenate([hcg, h_geom0], axis=-1),
                  fw1, fb1, fg1, fbe1, fw2, fb2, fg2, fbe2)
    return (h_geom, h_chem)


# speedup vs baseline: 9.1766x; 9.1766x over previous
import functools

import jax
import jax.numpy as jnp
import numpy as np
from jax.experimental import pallas as pl

B = 16
K = 16
H = 128
NGDF = 16
EPS = 1e-5


_SIG_D = np.float32(8.0 / NGDF)
_SIG_A = np.float32(2.0 / NGDF)


def _mu_row(d_min, d_max):
    step = (d_max - d_min) / (NGDF - 1)
    i = jax.lax.broadcasted_iota(jnp.int32, (1, NGDF), 1).astype(jnp.float32)
    return d_min + i * step


def _edge_h1(g1, g2, vn, w_ref, b1_ref, vt, et):
    vn_e = jnp.broadcast_to(vn[:, None, :], (vt, K, 8)).reshape(et, 8)
    dvec = g2[:, 0:3] - vn_e[:, 0:3]
    d2 = jnp.sum(dvec * dvec, axis=1, keepdims=True)
    dist = jnp.sqrt(d2)
    ang = jnp.sum(dvec * vn_e[:, 3:6], axis=1, keepdims=True) / dist
    mu_d = _mu_row(0.0, 8.0)
    mu_a = _mu_row(-1.0, 1.0)
    enc_d = jnp.exp(-(((dist - mu_d) / _SIG_D) ** 2))
    enc_a = jnp.exp(-(((ang - mu_a) / _SIG_A) ** 2))
    enc = jnp.concatenate([enc_d, enc_a], axis=1)
    return g1 + jnp.dot(enc, w_ref[...], preferred_element_type=jnp.float32) + b1_ref[...]


def _e1_body(g1_ref, g2_ref, vn_ref, w_ref, b1_ref, out_ref, *, vt, et):
    i = pl.program_id(0)

    @pl.when(i == 0)
    def _():
        out_ref[...] = jnp.zeros_like(out_ref)

    h1 = _edge_h1(g1_ref[...], g2_ref[...], vn_ref[...], w_ref, b1_ref, vt, et)
    out_ref[0:1, :] += jnp.sum(h1, axis=0, keepdims=True)
    out_ref[1:2, :] += jnp.sum(h1 * h1, axis=0, keepdims=True)


def _e2_body(g1_ref, g2_ref, vn_ref, w_ref, b1_ref, a1_ref, c1_ref,
             w2_ref, b2_ref, out_ref, *, vt, et):
    i = pl.program_id(0)

    @pl.when(i == 0)
    def _():
        out_ref[...] = jnp.zeros_like(out_ref)

    h1 = _edge_h1(g1_ref[...], g2_ref[...], vn_ref[...], w_ref, b1_ref, vt, et)
    h = h1 * a1_ref[...] + c1_ref[...]
    h = h * jax.nn.sigmoid(h)
    h2 = jnp.dot(h, w2_ref[...], preferred_element_type=jnp.float32) + b2_ref[...]
    out_ref[0:1, :] += jnp.sum(h2, axis=0, keepdims=True)
    out_ref[1:2, :] += jnp.sum(h2 * h2, axis=0, keepdims=True)


def _e3_body(g1_ref, g2_ref, vn_ref, w_ref, b1_ref, a1_ref, c1_ref,
             w2_ref, b2_ref, a2_ref, c2_ref, out_ref, *, vt, et):
    h1 = _edge_h1(g1_ref[...], g2_ref[...], vn_ref[...], w_ref, b1_ref, vt, et)
    h = h1 * a1_ref[...] + c1_ref[...]
    h = h * jax.nn.sigmoid(h)
    h2 = jnp.dot(h, w2_ref[...], preferred_element_type=jnp.float32) + b2_ref[...]
    y = h2 * a2_ref[...] + c2_ref[...]
    f2 = y[:, :H]
    c2 = y[:, H:]
    glu = jax.nn.sigmoid(f2) * jax.nn.softplus(c2)
    out_ref[...] = jnp.sum(glu.reshape(vt, K, H), axis=1)


def _knn_body(vp_ref, ppt_ref, out_ref, *, rt, npad):
    v = vp_ref[0]
    pt = ppt_ref[0]
    v4 = jnp.concatenate([-2.0 * v, jnp.ones((rt, 1), jnp.float32)], axis=1)
    s = jnp.dot(v4, pt, preferred_element_type=jnp.float32,
                precision=jax.lax.Precision.HIGHEST)
    iota = jax.lax.broadcasted_iota(jnp.int32, (rt, npad), 1)
    big = jnp.int32(2 ** 30)
    cols = []
    for _ in range(K):
        m = jnp.min(s, axis=1, keepdims=True)
        hit = s <= m
        cols.append(jnp.min(jnp.where(hit, iota, big), axis=1, keepdims=True))
        s = jnp.where(hit, jnp.inf, s)
    out_ref[0] = jnp.concatenate(cols, axis=1)


def _knn(verts, node_pos, n):
    npad = -(-n // 128) * 128
    rt = 640 if npad % 640 == 0 else 128
    vp = jnp.pad(verts.reshape(B, n, 3), ((0, 0), (0, npad - n), (0, 0)))
    pp = jnp.pad(node_pos.reshape(B, n, 3), ((0, 0), (0, npad - n), (0, 0)),
                 constant_values=1e6)
    ppt = jnp.transpose(pp, (0, 2, 1))
    ppt = jnp.concatenate(
        [ppt, jnp.sum(ppt * ppt, axis=1, keepdims=True)], axis=1)
    idx = pl.pallas_call(
        functools.partial(_knn_body, rt=rt, npad=npad),
        grid=(B, npad // rt),
        in_specs=[
            pl.BlockSpec((1, rt, 3), lambda b, r: (b, r, 0)),
            pl.BlockSpec((1, 4, npad), lambda b, r: (b, 0, 0)),
        ],
        out_specs=pl.BlockSpec((1, rt, K), lambda b, r: (b, r, 0)),
        out_shape=jax.ShapeDtypeStruct((B, npad, K), jnp.int32),
    )(vp, ppt)
    idx = idx[:, :n, :]
    idxf = (idx + (jnp.arange(B, dtype=jnp.int32) * n)[:, None, None]).reshape(-1)
    return idxf


def _bn_coefs(s, sq, count, g, be):
    mean = s / count
    var = sq / count - mean * mean
    a = g / jnp.sqrt(var + EPS)
    c = be - mean * a
    return a[None, :], c[None, :]


def _pick_vt(n_verts):
    for vt in (400, 200, 80, 40, 8):
        if n_verts % vt == 0:
            return vt
    return 8


def _edge_stage(z, g2, vn, sw1r, sb1, sg1, sbe1, sw2, sb2, sg2, sbe2, idxf):
    n_verts = vn.shape[0]
    e = idxf.shape[0]
    vt = _pick_vt(n_verts)
    et = vt * K
    grid = n_verts // vt

    g1 = z[idxf]

    espec = [
        pl.BlockSpec((et, H), lambda i: (i, 0)),
        pl.BlockSpec((et, 16), lambda i: (i, 0)),
        pl.BlockSpec((vt, 8), lambda i: (i, 0)),
        pl.BlockSpec((2 * NGDF, H), lambda i: (0, 0)),
        pl.BlockSpec((1, H), lambda i: (0, 0)),
    ]
    acc1 = pl.pallas_call(
        functools.partial(_e1_body, vt=vt, et=et),
        grid=(grid,),
        in_specs=espec,
        out_specs=pl.BlockSpec((2, H), lambda i: (0, 0)),
        out_shape=jax.ShapeDtypeStruct((2, H), jnp.float32),
    )(g1, g2, vn, sw1r, sb1[None, :])
    a1, c1 = _bn_coefs(acc1[0], acc1[1], e, sg1, sbe1)

    espec2 = espec + [
        pl.BlockSpec((1, H), lambda i: (0, 0)),
        pl.BlockSpec((1, H), lambda i: (0, 0)),
        pl.BlockSpec((H, 2 * H), lambda i: (0, 0)),
        pl.BlockSpec((1, 2 * H), lambda i: (0, 0)),
    ]
    acc2 = pl.pallas_call(
        functools.partial(_e2_body, vt=vt, et=et),
        grid=(grid,),
        in_specs=espec2,
        out_specs=pl.BlockSpec((2, 2 * H), lambda i: (0, 0)),
        out_shape=jax.ShapeDtypeStruct((2, 2 * H), jnp.float32),
    )(g1, g2, vn, sw1r, sb1[None, :], a1, c1, sw2, sb2[None, :])
    a2, c2 = _bn_coefs(acc2[0], acc2[1], e, sg2, sbe2)

    espec3 = espec2 + [
        pl.BlockSpec((1, 2 * H), lambda i: (0, 0)),
        pl.BlockSpec((1, 2 * H), lambda i: (0, 0)),
    ]
    hcg = pl.pallas_call(
        functools.partial(_e3_body, vt=vt, et=et),
        grid=(grid,),
        in_specs=espec3,
        out_specs=pl.BlockSpec((vt, H), lambda i: (i, 0)),
        out_shape=jax.ShapeDtypeStruct((n_verts, H), jnp.float32),
    )(g1, g2, vn, sw1r, sb1[None, :], a1, c1, sw2, sb2[None, :], a2, c2)
    return hcg


def _bn(x, g, b):
    m = jnp.mean(x, axis=0)
    v = jnp.var(x, axis=0)
    return g * (x - m) / jnp.sqrt(v + EPS) + b


def _mlp(x, w1, b1, g1, be1, w2, b2, g2, be2):
    h = x @ w1 + b1
    h = _bn(h, g1, be1)
    h = jax.nn.silu(h)
    h = h @ w2 + b2
    h = _bn(h, g2, be2)
    return h


def kernel(graph_x, surface_x, verts, node_pos, vnormals,
           cw1, cb1, cg1, cbe1, cw2, cb2, cg2, cbe2,
           gw1, gb1, gg1, gbe1, gw2, gb2, gg2, gbe2,
           sw1, sb1, sg1, sbe1, sw2, sb2, sg2, sbe2,
           fw1, fb1, fg1, fbe1, fw2, fb2, fg2, fbe2):
    n_total = verts.shape[0]
    n = n_total // B

    h_geom0 = _mlp(surface_x, gw1, gb1, gg1, gbe1, gw2, gb2, gg2, gbe2)
    h_chem2 = _mlp(graph_x, cw1, cb1, cg1, cbe1, cw2, cb2, cg2, cbe2)
    f, c = jnp.split(h_chem2, 2, axis=-1)
    h_chem = jax.nn.sigmoid(f) * jax.nn.softplus(c)

    idxf = _knn(verts, node_pos, n)

    z = h_chem @ sw1[:H]
    e = idxf.shape[0]
    g2 = jnp.concatenate(
        [node_pos[idxf], jnp.zeros((e, 13), jnp.float32)], axis=1)
    vn = jnp.concatenate(
        [verts, vnormals, jnp.zeros((n_total, 2), jnp.float32)], axis=1)

    hcg = _edge_stage(z, g2, vn, sw1[H:], sb1, sg1, sbe1,
                      sw2, sb2, sg2, sbe2, idxf)

    h_geom = _mlp(jnp.concatenate([hcg, h_geom0], axis=-1),
                  fw1, fb1, fg1, fbe1, fw2, fb2, fg2, fbe2)
    return (h_geom, h_chem)


# speedup vs baseline: 13.0422x; 1.4212x over previous
import functools

import jax
import jax.numpy as jnp
import numpy as np
from jax import lax
from jax.experimental import pallas as pl
from jax.experimental.pallas import tpu as pltpu
from jax.experimental.pallas import tpu_sc as plsc

B = 16
K = 16
H = 128
NGDF = 16
EPS = 1e-5


_SIG_D = np.float32(8.0 / NGDF)
_SIG_A = np.float32(2.0 / NGDF)


def _mu_row(d_min, d_max):
    step = (d_max - d_min) / (NGDF - 1)
    i = jax.lax.broadcasted_iota(jnp.int32, (1, NGDF), 1).astype(jnp.float32)
    return d_min + i * step


def _edge_h1(g, geo, w_ref, b1_ref, vt, et):
    dist = geo[:, 0:1]
    ang = geo[:, 1:2]
    mu_d = _mu_row(0.0, 8.0)
    mu_a = _mu_row(-1.0, 1.0)
    enc_d = jnp.exp(-(((dist - mu_d) / _SIG_D) ** 2))
    enc_a = jnp.exp(-(((ang - mu_a) / _SIG_A) ** 2))
    enc = jnp.concatenate([enc_d, enc_a], axis=1)
    return (g
            + jnp.dot(enc, w_ref[...], preferred_element_type=jnp.float32)
            + b1_ref[...])


def _e1_body(g_ref, geo_ref, w_ref, b1_ref, out_ref, *, vt, et):
    i = pl.program_id(0)

    @pl.when(i == 0)
    def _():
        out_ref[...] = jnp.zeros_like(out_ref)

    h1 = _edge_h1(g_ref[...], geo_ref[...], w_ref, b1_ref, vt, et)
    out_ref[0:1, :] += jnp.sum(h1, axis=0, keepdims=True)
    out_ref[1:2, :] += jnp.sum(h1 * h1, axis=0, keepdims=True)


def _e2_body(g_ref, geo_ref, w_ref, b1_ref, a1_ref, c1_ref,
             w2_ref, b2_ref, out_ref, *, vt, et):
    i = pl.program_id(0)

    @pl.when(i == 0)
    def _():
        out_ref[...] = jnp.zeros_like(out_ref)

    h1 = _edge_h1(g_ref[...], geo_ref[...], w_ref, b1_ref, vt, et)
    h = h1 * a1_ref[...] + c1_ref[...]
    h = h * jax.nn.sigmoid(h)
    h2 = jnp.dot(h, w2_ref[...], preferred_element_type=jnp.float32) + b2_ref[...]
    out_ref[0:1, :] += jnp.sum(h2, axis=0, keepdims=True)
    out_ref[1:2, :] += jnp.sum(h2 * h2, axis=0, keepdims=True)


def _e3_body(g_ref, geo_ref, w_ref, b1_ref, a1_ref, c1_ref,
             w2_ref, b2_ref, a2_ref, c2_ref, out_ref, *, vt, et):
    h1 = _edge_h1(g_ref[...], geo_ref[...], w_ref, b1_ref, vt, et)
    h = h1 * a1_ref[...] + c1_ref[...]
    h = h * jax.nn.sigmoid(h)
    h2 = jnp.dot(h, w2_ref[...], preferred_element_type=jnp.float32) + b2_ref[...]
    y = h2 * a2_ref[...] + c2_ref[...]
    f2 = y[:, :H]
    c2 = y[:, H:]
    glu = jax.nn.sigmoid(f2) * jax.nn.softplus(c2)
    out_ref[...] = jnp.sum(glu.reshape(vt, K, H), axis=1)


def _knn_body(vp_ref, nvp_ref, ppt_ref, idx_ref, d_ref, a_ref, *, rt, npad):
    v = vp_ref[0]
    nv = nvp_ref[0]
    pt = ppt_ref[0]
    v4 = jnp.concatenate([-2.0 * v, jnp.ones((rt, 1), jnp.float32)], axis=1)
    s = jnp.dot(v4, pt, preferred_element_type=jnp.float32,
                precision=jax.lax.Precision.HIGHEST)
    q = jnp.dot(nv, pt[0:3, :], preferred_element_type=jnp.float32,
                precision=jax.lax.Precision.HIGHEST)
    vn2 = jnp.sum(v * v, axis=1, keepdims=True)
    vdn = jnp.sum(v * nv, axis=1, keepdims=True)
    iota = jax.lax.broadcasted_iota(jnp.int32, (rt, npad), 1)
    big = jnp.int32(2 ** 30)
    bigf = jnp.float32(3e38)
    icols, dcols, acols = [], [], []
    for _ in range(K):
        m = jnp.min(s, axis=1, keepdims=True)
        hit = s <= m
        icols.append(jnp.min(jnp.where(hit, iota, big), axis=1, keepdims=True))
        qs = jnp.min(jnp.where(hit, q, bigf), axis=1, keepdims=True)
        d = jnp.sqrt(jnp.maximum(m + vn2, 0.0))
        dcols.append(d)
        acols.append((qs - vdn) / d)
        s = jnp.where(hit, jnp.inf, s)
    idx_ref[0] = jnp.concatenate(icols, axis=1)
    d_ref[0] = jnp.concatenate(dcols, axis=1)
    a_ref[0] = jnp.concatenate(acols, axis=1)


def _knn(verts, node_pos, vnormals, n):
    npad = -(-n // 128) * 128
    rt = 128
    vp = jnp.pad(verts.reshape(B, n, 3), ((0, 0), (0, npad - n), (0, 0)))
    nvp = jnp.pad(vnormals.reshape(B, n, 3), ((0, 0), (0, npad - n), (0, 0)))
    pp = jnp.pad(node_pos.reshape(B, n, 3), ((0, 0), (0, npad - n), (0, 0)),
                 constant_values=1e6)
    ppt = jnp.transpose(pp, (0, 2, 1))
    ppt = jnp.concatenate(
        [ppt, jnp.sum(ppt * ppt, axis=1, keepdims=True)], axis=1)
    idx, d, a = pl.pallas_call(
        functools.partial(_knn_body, rt=rt, npad=npad),
        grid=(B, npad // rt),
        in_specs=[
            pl.BlockSpec((1, rt, 3), lambda b, r: (b, r, 0)),
            pl.BlockSpec((1, rt, 3), lambda b, r: (b, r, 0)),
            pl.BlockSpec((1, 4, npad), lambda b, r: (b, 0, 0)),
        ],
        out_specs=[
            pl.BlockSpec((1, rt, K), lambda b, r: (b, r, 0)),
            pl.BlockSpec((1, rt, K), lambda b, r: (b, r, 0)),
            pl.BlockSpec((1, rt, K), lambda b, r: (b, r, 0)),
        ],
        out_shape=[
            jax.ShapeDtypeStruct((B, npad, K), jnp.int32),
            jax.ShapeDtypeStruct((B, npad, K), jnp.float32),
            jax.ShapeDtypeStruct((B, npad, K), jnp.float32),
        ],
    )(vp, nvp, ppt)
    idx = idx[:, :n, :]
    idxf = (idx + (jnp.arange(B, dtype=jnp.int32) * n)[:, None, None]).reshape(-1)
    e = idxf.shape[0]
    dist_e = d[:, :n, :].reshape(e, 1)
    ang_e = a[:, :n, :].reshape(e, 1)
    geo = jnp.concatenate(
        [dist_e, ang_e, jnp.zeros((e, 14), jnp.float32)], axis=1)
    return idxf, geo


_NW = 32
_GC = 128


def _sc_gather(table, idxf):
    e = idxf.shape[0]
    td = table.shape[1]
    nchunks = e // _GC
    trips = -(-nchunks // _NW)
    mesh = plsc.VectorSubcoreMesh(core_axis_name="c", subcore_axis_name="s")

    def body(t_hbm, idx_hbm, out_hbm, idx_v, rows_v, sem):
        wid = lax.axis_index("c") * 16 + lax.axis_index("s")

        def step(i, _):
            g = wid + i * _NW

            @pl.when(g < nchunks)
            def _():
                base = g * _GC
                pltpu.sync_copy(idx_hbm.at[pl.ds(base, _GC)], idx_v)
                pltpu.async_copy(t_hbm.at[idx_v], rows_v, sem).wait()
                pltpu.sync_copy(rows_v, out_hbm.at[pl.ds(base, _GC)])
            return 0

        lax.fori_loop(0, trips, step, 0)

    return pl.kernel(
        body,
        out_type=jax.ShapeDtypeStruct((e, td), jnp.float32),
        mesh=mesh,
        scratch_types=[
            pltpu.VMEM((_GC,), jnp.int32),
            pltpu.VMEM((_GC, td), jnp.float32),
            pltpu.SemaphoreType.DMA,
        ],
    )(table, idxf)


def _bn_coefs(s, sq, count, g, be):
    mean = s / count
    var = sq / count - mean * mean
    a = g / jnp.sqrt(var + EPS)
    c = be - mean * a
    return a[None, :], c[None, :]


def _pick_vt(n_verts):
    for vt in (400, 200, 80, 40, 8):
        if n_verts % vt == 0:
            return vt
    return 8


def _edge_stage(gath, geo, n_verts, sw1r, sb1, sg1, sbe1, sw2, sb2, sg2, sbe2):
    e = gath.shape[0]
    vt = _pick_vt(n_verts)
    et = vt * K
    grid = n_verts // vt

    espec = [
        pl.BlockSpec((et, H), lambda i: (i, 0)),
        pl.BlockSpec((et, 16), lambda i: (i, 0)),
        pl.BlockSpec((2 * NGDF, H), lambda i: (0, 0)),
        pl.BlockSpec((1, H), lambda i: (0, 0)),
    ]
    acc1 = pl.pallas_call(
        functools.partial(_e1_body, vt=vt, et=et),
        grid=(grid,),
        in_specs=espec,
        out_specs=pl.BlockSpec((2, H), lambda i: (0, 0)),
        out_shape=jax.ShapeDtypeStruct((2, H), jnp.float32),
    )(gath, geo, sw1r, sb1[None, :])
    a1, c1 = _bn_coefs(acc1[0], acc1[1], e, sg1, sbe1)

    espec2 = espec + [
        pl.BlockSpec((1, H), lambda i: (0, 0)),
        pl.BlockSpec((1, H), lambda i: (0, 0)),
        pl.BlockSpec((H, 2 * H), lambda i: (0, 0)),
        pl.BlockSpec((1, 2 * H), lambda i: (0, 0)),
    ]
    acc2 = pl.pallas_call(
        functools.partial(_e2_body, vt=vt, et=et),
        grid=(grid,),
        in_specs=espec2,
        out_specs=pl.BlockSpec((2, 2 * H), lambda i: (0, 0)),
        out_shape=jax.ShapeDtypeStruct((2, 2 * H), jnp.float32),
    )(gath, geo, sw1r, sb1[None, :], a1, c1, sw2, sb2[None, :])
    a2, c2 = _bn_coefs(acc2[0], acc2[1], e, sg2, sbe2)

    espec3 = espec2 + [
        pl.BlockSpec((1, 2 * H), lambda i: (0, 0)),
        pl.BlockSpec((1, 2 * H), lambda i: (0, 0)),
    ]
    hcg = pl.pallas_call(
        functools.partial(_e3_body, vt=vt, et=et),
        grid=(grid,),
        in_specs=espec3,
        out_specs=pl.BlockSpec((vt, H), lambda i: (i, 0)),
        out_shape=jax.ShapeDtypeStruct((n_verts, H), jnp.float32),
    )(gath, geo, sw1r, sb1[None, :], a1, c1, sw2, sb2[None, :], a2, c2)
    return hcg


def _bn(x, g, b):
    m = jnp.mean(x, axis=0)
    v = jnp.var(x, axis=0)
    return g * (x - m) / jnp.sqrt(v + EPS) + b


def _mlp(x, w1, b1, g1, be1, w2, b2, g2, be2):
    h = x @ w1 + b1
    h = _bn(h, g1, be1)
    h = jax.nn.silu(h)
    h = h @ w2 + b2
    h = _bn(h, g2, be2)
    return h


def kernel(graph_x, surface_x, verts, node_pos, vnormals,
           cw1, cb1, cg1, cbe1, cw2, cb2, cg2, cbe2,
           gw1, gb1, gg1, gbe1, gw2, gb2, gg2, gbe2,
           sw1, sb1, sg1, sbe1, sw2, sb2, sg2, sbe2,
           fw1, fb1, fg1, fbe1, fw2, fb2, fg2, fbe2):
    n_total = verts.shape[0]
    n = n_total // B

    h_geom0 = _mlp(surface_x, gw1, gb1, gg1, gbe1, gw2, gb2, gg2, gbe2)
    h_chem2 = _mlp(graph_x, cw1, cb1, cg1, cbe1, cw2, cb2, cg2, cbe2)
    f, c = jnp.split(h_chem2, 2, axis=-1)
    h_chem = jax.nn.sigmoid(f) * jax.nn.softplus(c)

    idxf, geo = _knn(verts, node_pos, vnormals, n)

    z = h_chem @ sw1[:H]
    gath = _sc_gather(z, idxf)
    hcg = _edge_stage(gath, geo, n_total, sw1[H:], sb1, sg1, sbe1,
                      sw2, sb2, sg2, sbe2)

    h_geom = _mlp(jnp.concatenate([hcg, h_geom0], axis=-1),
                  fw1, fb1, fg1, fbe1, fw2, fb2, fg2, fbe2)
    return (h_geom, h_chem)


# speedup vs baseline: 13.0699x; 1.0021x over previous
import functools

import jax
import jax.numpy as jnp
import numpy as np
from jax import lax
from jax.experimental import pallas as pl
from jax.experimental.pallas import tpu as pltpu
from jax.experimental.pallas import tpu_sc as plsc

B = 16
K = 16
H = 128
NGDF = 16
EPS = 1e-5


_SIG_D = np.float32(8.0 / NGDF)
_SIG_A = np.float32(2.0 / NGDF)


def _mu_row(d_min, d_max):
    step = (d_max - d_min) / (NGDF - 1)
    i = jax.lax.broadcasted_iota(jnp.int32, (1, NGDF), 1).astype(jnp.float32)
    return d_min + i * step


def _edge_h1(g, geo, w_ref, b1_ref, vt, et):
    dist = geo[:, 0:1]
    ang = geo[:, 1:2]
    mu_d = _mu_row(0.0, 8.0)
    mu_a = _mu_row(-1.0, 1.0)
    enc_d = jnp.exp(-(((dist - mu_d) / _SIG_D) ** 2))
    enc_a = jnp.exp(-(((ang - mu_a) / _SIG_A) ** 2))
    enc = jnp.concatenate([enc_d, enc_a], axis=1)
    return (g
            + jnp.dot(enc, w_ref[...], preferred_element_type=jnp.float32)
            + b1_ref[...])


def _e1_body(g_ref, geo_ref, w_ref, b1_ref, out_ref, *, vt, et):
    i = pl.program_id(0)

    @pl.when(i == 0)
    def _():
        out_ref[...] = jnp.zeros_like(out_ref)

    h1 = _edge_h1(g_ref[...], geo_ref[...], w_ref, b1_ref, vt, et)
    out_ref[0:1, :] += jnp.sum(h1, axis=0, keepdims=True)
    out_ref[1:2, :] += jnp.sum(h1 * h1, axis=0, keepdims=True)


def _e2_body(g_ref, geo_ref, w_ref, b1_ref, a1_ref, c1_ref,
             w2_ref, b2_ref, out_ref, *, vt, et):
    i = pl.program_id(0)

    @pl.when(i == 0)
    def _():
        out_ref[...] = jnp.zeros_like(out_ref)

    h1 = _edge_h1(g_ref[...], geo_ref[...], w_ref, b1_ref, vt, et)
    h = h1 * a1_ref[...] + c1_ref[...]
    h = h * jax.nn.sigmoid(h)
    h2 = jnp.dot(h, w2_ref[...], preferred_element_type=jnp.float32) + b2_ref[...]
    out_ref[0:1, :] += jnp.sum(h2, axis=0, keepdims=True)
    out_ref[1:2, :] += jnp.sum(h2 * h2, axis=0, keepdims=True)


def _e3_body(g_ref, geo_ref, w_ref, b1_ref, a1_ref, c1_ref,
             w2_ref, b2_ref, a2_ref, c2_ref, out_ref, *, vt, et):
    h1 = _edge_h1(g_ref[...], geo_ref[...], w_ref, b1_ref, vt, et)
    h = h1 * a1_ref[...] + c1_ref[...]
    h = h * jax.nn.sigmoid(h)
    h2 = jnp.dot(h, w2_ref[...], preferred_element_type=jnp.float32) + b2_ref[...]
    y = h2 * a2_ref[...] + c2_ref[...]
    f2 = y[:, :H]
    c2 = y[:, H:]
    glu = jax.nn.sigmoid(f2) * jax.nn.softplus(c2)
    out_ref[...] = jnp.sum(glu.reshape(vt, K, H), axis=1)


def _knn_body(vp_ref, nvp_ref, ppt_ref, idx_ref, d_ref, a_ref, *, rt, npad):
    v = vp_ref[0]
    nv = nvp_ref[0]
    pt = ppt_ref[0]
    v4 = jnp.concatenate([-2.0 * v, jnp.ones((rt, 1), jnp.float32)], axis=1)
    s = jnp.dot(v4, pt, preferred_element_type=jnp.float32,
                precision=jax.lax.Precision.HIGHEST)
    q = jnp.dot(nv, pt[0:3, :], preferred_element_type=jnp.float32,
                precision=jax.lax.Precision.HIGHEST)
    vn2 = jnp.sum(v * v, axis=1, keepdims=True)
    vdn = jnp.sum(v * nv, axis=1, keepdims=True)
    iota = jax.lax.broadcasted_iota(jnp.int32, (rt, npad), 1)
    big = jnp.int32(2 ** 30)
    bigf = jnp.float32(3e38)
    icols, dcols, acols = [], [], []
    for _ in range(K):
        m = jnp.min(s, axis=1, keepdims=True)
        hit = s <= m
        icols.append(jnp.min(jnp.where(hit, iota, big), axis=1, keepdims=True))
        qs = jnp.min(jnp.where(hit, q, bigf), axis=1, keepdims=True)
        d = jnp.sqrt(jnp.maximum(m + vn2, 0.0))
        dcols.append(d)
        acols.append((qs - vdn) / d)
        s = jnp.where(hit, jnp.inf, s)
    idx_ref[0] = jnp.concatenate(icols, axis=1)
    d_ref[0] = jnp.concatenate(dcols, axis=1)
    a_ref[0] = jnp.concatenate(acols, axis=1)


def _knn(verts, node_pos, vnormals, n):
    npad = -(-n // 128) * 128
    rt = 128
    vp = jnp.pad(verts.reshape(B, n, 3), ((0, 0), (0, npad - n), (0, 0)))
    nvp = jnp.pad(vnormals.reshape(B, n, 3), ((0, 0), (0, npad - n), (0, 0)))
    pp = jnp.pad(node_pos.reshape(B, n, 3), ((0, 0), (0, npad - n), (0, 0)),
                 constant_values=1e6)
    ppt = jnp.transpose(pp, (0, 2, 1))
    ppt = jnp.concatenate(
        [ppt, jnp.sum(ppt * ppt, axis=1, keepdims=True)], axis=1)
    idx, d, a = pl.pallas_call(
        functools.partial(_knn_body, rt=rt, npad=npad),
        grid=(B, npad // rt),
        in_specs=[
            pl.BlockSpec((1, rt, 3), lambda b, r: (b, r, 0)),
            pl.BlockSpec((1, rt, 3), lambda b, r: (b, r, 0)),
            pl.BlockSpec((1, 4, npad), lambda b, r: (b, 0, 0)),
        ],
        out_specs=[
            pl.BlockSpec((1, rt, K), lambda b, r: (b, r, 0)),
            pl.BlockSpec((1, rt, K), lambda b, r: (b, r, 0)),
            pl.BlockSpec((1, rt, K), lambda b, r: (b, r, 0)),
        ],
        out_shape=[
            jax.ShapeDtypeStruct((B, npad, K), jnp.int32),
            jax.ShapeDtypeStruct((B, npad, K), jnp.float32),
            jax.ShapeDtypeStruct((B, npad, K), jnp.float32),
        ],
    )(vp, nvp, ppt)
    idx = idx[:, :n, :]
    idxf = (idx + (jnp.arange(B, dtype=jnp.int32) * n)[:, None, None]).reshape(-1)
    e = idxf.shape[0]
    dist_e = d[:, :n, :].reshape(e, 1)
    ang_e = a[:, :n, :].reshape(e, 1)
    geo = jnp.concatenate(
        [dist_e, ang_e, jnp.zeros((e, 14), jnp.float32)], axis=1)
    return idxf, geo


_NW = 32
_GC = 128


def _sc_gather(table, idxf):
    e = idxf.shape[0]
    td = table.shape[1]
    nchunks = e // _GC
    trips = -(-nchunks // _NW)
    mesh = plsc.VectorSubcoreMesh(core_axis_name="c", subcore_axis_name="s")

    def body(t_hbm, idx_hbm, out_hbm, idx_v, rows_v, sem):
        wid = lax.axis_index("c") * 16 + lax.axis_index("s")

        def step(i, _):
            g = wid + i * _NW

            @pl.when(g < nchunks)
            def _():
                base = g * _GC
                pltpu.sync_copy(idx_hbm.at[pl.ds(base, _GC)], idx_v)
                pltpu.async_copy(t_hbm.at[idx_v], rows_v, sem).wait()
                pltpu.sync_copy(rows_v, out_hbm.at[pl.ds(base, _GC)])
            return 0

        lax.fori_loop(0, trips, step, 0)

    return pl.kernel(
        body,
        out_type=jax.ShapeDtypeStruct((e, td), jnp.float32),
        mesh=mesh,
        scratch_types=[
            pltpu.VMEM((_GC,), jnp.int32),
            pltpu.VMEM((_GC, td), jnp.float32),
            pltpu.SemaphoreType.DMA,
        ],
    )(table, idxf)


def _bn_coefs(s, sq, count, g, be):
    mean = s / count
    var = sq / count - mean * mean
    a = g / jnp.sqrt(var + EPS)
    c = be - mean * a
    return a[None, :], c[None, :]


def _pick_vt(n_verts):
    for vt in (400, 200, 80, 40, 8):
        if n_verts % vt == 0:
            return vt
    return 8


def _edge_stage(gath, geo, n_verts, sw1r, sb1, sg1, sbe1, sw2, sb2, sg2, sbe2):
    e = gath.shape[0]
    vt = _pick_vt(n_verts)
    et = vt * K
    grid = n_verts // vt

    espec = [
        pl.BlockSpec((et, H), lambda i: (i, 0)),
        pl.BlockSpec((et, 16), lambda i: (i, 0)),
        pl.BlockSpec((2 * NGDF, H), lambda i: (0, 0)),
        pl.BlockSpec((1, H), lambda i: (0, 0)),
    ]
    acc1 = pl.pallas_call(
        functools.partial(_e1_body, vt=vt, et=et),
        grid=(grid,),
        in_specs=espec,
        out_specs=pl.BlockSpec((2, H), lambda i: (0, 0)),
        out_shape=jax.ShapeDtypeStruct((2, H), jnp.float32),
    )(gath, geo, sw1r, sb1[None, :])
    a1, c1 = _bn_coefs(acc1[0], acc1[1], e, sg1, sbe1)

    espec2 = espec + [
        pl.BlockSpec((1, H), lambda i: (0, 0)),
        pl.BlockSpec((1, H), lambda i: (0, 0)),
        pl.BlockSpec((H, 2 * H), lambda i: (0, 0)),
        pl.BlockSpec((1, 2 * H), lambda i: (0, 0)),
    ]
    acc2 = pl.pallas_call(
        functools.partial(_e2_body, vt=vt, et=et),
        grid=(grid,),
        in_specs=espec2,
        out_specs=pl.BlockSpec((2, 2 * H), lambda i: (0, 0)),
        out_shape=jax.ShapeDtypeStruct((2, 2 * H), jnp.float32),
    )(gath, geo, sw1r, sb1[None, :], a1, c1, sw2, sb2[None, :])
    a2, c2 = _bn_coefs(acc2[0], acc2[1], e, sg2, sbe2)

    espec3 = espec2 + [
        pl.BlockSpec((1, 2 * H), lambda i: (0, 0)),
        pl.BlockSpec((1, 2 * H), lambda i: (0, 0)),
    ]
    hcg = pl.pallas_call(
        functools.partial(_e3_body, vt=vt, et=et),
        grid=(grid,),
        in_specs=espec3,
        out_specs=pl.BlockSpec((vt, H), lambda i: (i, 0)),
        out_shape=jax.ShapeDtypeStruct((n_verts, H), jnp.float32),
    )(gath, geo, sw1r, sb1[None, :], a1, c1, sw2, sb2[None, :], a2, c2)
    return hcg


def _acc_stats(s_ref, y, first):
    @pl.when(first)
    def _():
        s_ref[...] = jnp.zeros_like(s_ref)

    s_ref[0:1, :] += jnp.sum(y, axis=0, keepdims=True)
    s_ref[1:2, :] += jnp.sum(y * y, axis=0, keepdims=True)


def _lin_body(x_ref, w_ref, b_ref, y_ref, s_ref):
    y = jnp.dot(x_ref[...], w_ref[...],
                preferred_element_type=jnp.float32) + b_ref[...]
    y_ref[...] = y
    _acc_stats(s_ref, y, pl.program_id(0) == 0)


def _lin2_body(y_ref, a_ref, c_ref, w_ref, b_ref, y2_ref, s_ref):
    h = y_ref[...] * a_ref[...] + c_ref[...]
    h = h * jax.nn.sigmoid(h)
    y2 = jnp.dot(h, w_ref[...],
                 preferred_element_type=jnp.float32) + b_ref[...]
    y2_ref[...] = y2
    _acc_stats(s_ref, y2, pl.program_id(0) == 0)


def _chem_body(y_ref, a_ref, c_ref, w_ref, hc_ref, z_ref):
    yb = y_ref[...] * a_ref[...] + c_ref[...]
    hc = jax.nn.sigmoid(yb[:, :H]) * jax.nn.softplus(yb[:, H:])
    hc_ref[...] = hc
    z_ref[...] = jnp.dot(hc, w_ref[...], preferred_element_type=jnp.float32)


def _f1_body(hcg_ref, yg_ref, ag_ref, cg_ref, w_ref, b_ref, y_ref, s_ref):
    hg = yg_ref[...] * ag_ref[...] + cg_ref[...]
    x = jnp.concatenate([hcg_ref[...], hg], axis=1)
    y = jnp.dot(x, w_ref[...], preferred_element_type=jnp.float32) + b_ref[...]
    y_ref[...] = y
    _acc_stats(s_ref, y, pl.program_id(0) == 0)


def _bn_apply_body(y_ref, a_ref, c_ref, out_ref):
    out_ref[...] = y_ref[...] * a_ref[...] + c_ref[...]


def _pick_rows(n):
    for r in (2000, 400, 200, 80, 8):
        if n % r == 0:
            return r
    return 8


def _row_spec(rt, d):
    return pl.BlockSpec((rt, d), lambda i: (i, 0))


def _full_spec(r, d):
    return pl.BlockSpec((r, d), lambda i: (0, 0))


def _dense_call(body, n, rt, in_specs, outd, arrays, extra_outs=None):
    out_specs = [_row_spec(rt, outd), pl.BlockSpec((2, outd), lambda i: (0, 0))]
    out_shape = [jax.ShapeDtypeStruct((n, outd), jnp.float32),
                 jax.ShapeDtypeStruct((2, outd), jnp.float32)]
    return pl.pallas_call(
        body, grid=(n // rt,), in_specs=in_specs,
        out_specs=out_specs, out_shape=out_shape)(*arrays)


def _lin(x, w, b):
    n, ind = x.shape
    outd = w.shape[1]
    rt = _pick_rows(n)
    return _dense_call(
        _lin_body, n, rt,
        [_row_spec(rt, ind), _full_spec(ind, outd), _full_spec(1, outd)],
        outd, (x, w, b[None, :]))


def _lin2(y, a, c, w, b):
    n, ind = y.shape
    outd = w.shape[1]
    rt = _pick_rows(n)
    return _dense_call(
        _lin2_body, n, rt,
        [_row_spec(rt, ind), _full_spec(1, ind), _full_spec(1, ind),
         _full_spec(ind, outd), _full_spec(1, outd)],
        outd, (y, a, c, w, b[None, :]))


def _chem_apply(y, a, c, w):
    n = y.shape[0]
    rt = _pick_rows(n)
    return pl.pallas_call(
        _chem_body, grid=(n // rt,),
        in_specs=[_row_spec(rt, 2 * H), _full_spec(1, 2 * H),
                  _full_spec(1, 2 * H), _full_spec(H, H)],
        out_specs=[_row_spec(rt, H), _row_spec(rt, H)],
        out_shape=[jax.ShapeDtypeStruct((n, H), jnp.float32),
                   jax.ShapeDtypeStruct((n, H), jnp.float32)],
    )(y, a, c, w)


def _f1(hcg, yg, ag, cg, w, b):
    n = hcg.shape[0]
    rt = _pick_rows(n)
    return _dense_call(
        _f1_body, n, rt,
        [_row_spec(rt, H), _row_spec(rt, H), _full_spec(1, H),
         _full_spec(1, H), _full_spec(2 * H, H), _full_spec(1, H)],
        H, (hcg, yg, ag, cg, w, b[None, :]))


def _bn_apply(y, a, c):
    n, d = y.shape
    rt = _pick_rows(n)
    return pl.pallas_call(
        _bn_apply_body, grid=(n // rt,),
        in_specs=[_row_spec(rt, d), _full_spec(1, d), _full_spec(1, d)],
        out_specs=_row_spec(rt, d),
        out_shape=jax.ShapeDtypeStruct((n, d), jnp.float32),
    )(y, a, c)


def kernel(graph_x, surface_x, verts, node_pos, vnormals,
           cw1, cb1, cg1, cbe1, cw2, cb2, cg2, cbe2,
           gw1, gb1, gg1, gbe1, gw2, gb2, gg2, gbe2,
           sw1, sb1, sg1, sbe1, sw2, sb2, sg2, sbe2,
           fw1, fb1, fg1, fbe1, fw2, fb2, fg2, fbe2):
    n_total = verts.shape[0]
    n = n_total // B

    y_c1, st = _lin(graph_x, cw1, cb1)
    a, c = _bn_coefs(st[0], st[1], n_total, cg1, cbe1)
    y_c2, st = _lin2(y_c1, a, c, cw2, cb2)
    a, c = _bn_coefs(st[0], st[1], n_total, cg2, cbe2)
    h_chem, z = _chem_apply(y_c2, a, c, sw1[:H])

    y_g1, st = _lin(surface_x, gw1, gb1)
    a, c = _bn_coefs(st[0], st[1], n_total, gg1, gbe1)
    y_g2, st = _lin2(y_g1, a, c, gw2, gb2)
    ag2, cg2_ = _bn_coefs(st[0], st[1], n_total, gg2, gbe2)

    idxf, geo = _knn(verts, node_pos, vnormals, n)

    gath = _sc_gather(z, idxf)
    hcg = _edge_stage(gath, geo, n_total, sw1[H:], sb1, sg1, sbe1,
                      sw2, sb2, sg2, sbe2)

    y_f1, st = _f1(hcg, y_g2, ag2, cg2_, fw1, fb1)
    a, c = _bn_coefs(st[0], st[1], n_total, fg1, fbe1)
    y_f2, st = _lin2(y_f1, a, c, fw2, fb2)
    a, c = _bn_coefs(st[0], st[1], n_total, fg2, fbe2)
    h_geom = _bn_apply(y_f2, a, c)
    return (h_geom, h_chem)


# speedup vs baseline: 13.5089x; 1.0336x over previous
import functools

import jax
import jax.numpy as jnp
import numpy as np
from jax import lax
from jax.experimental import pallas as pl
from jax.experimental.pallas import tpu as pltpu
from jax.experimental.pallas import tpu_sc as plsc

B = 16
K = 16
H = 128
NGDF = 16
EPS = 1e-5


_SIG_D = np.float32(8.0 / NGDF)
_SIG_A = np.float32(2.0 / NGDF)


def _mu_row(d_min, d_max):
    step = (d_max - d_min) / (NGDF - 1)
    i = jax.lax.broadcasted_iota(jnp.int32, (1, NGDF), 1).astype(jnp.float32)
    return d_min + i * step


def _enc_body(geo_ref, out_ref):
    geo = geo_ref[...]
    dist = geo[:, 0:1]
    ang = geo[:, 1:2]
    mu_d = _mu_row(0.0, 8.0)
    mu_a = _mu_row(-1.0, 1.0)
    enc_d = jnp.exp(-(((dist - mu_d) / _SIG_D) ** 2))
    enc_a = jnp.exp(-(((ang - mu_a) / _SIG_A) ** 2))
    out_ref[...] = jnp.concatenate([enc_d, enc_a], axis=1)


def _edge_h1(g, enc, w_ref, b1_ref, vt, et):
    return (g
            + jnp.dot(enc, w_ref[...], preferred_element_type=jnp.float32)
            + b1_ref[...])


def _e1_body(g_ref, enc_ref, w_ref, b1_ref, out_ref, *, vt, et):
    i = pl.program_id(0)

    @pl.when(i == 0)
    def _():
        out_ref[...] = jnp.zeros_like(out_ref)

    h1 = _edge_h1(g_ref[...], enc_ref[...], w_ref, b1_ref, vt, et)
    out_ref[0:1, :] += jnp.sum(h1, axis=0, keepdims=True)
    out_ref[1:2, :] += jnp.sum(h1 * h1, axis=0, keepdims=True)


def _e2_body(g_ref, enc_ref, w_ref, b1_ref, a1_ref, c1_ref,
             w2_ref, b2_ref, out_ref, *, vt, et):
    i = pl.program_id(0)

    @pl.when(i == 0)
    def _():
        out_ref[...] = jnp.zeros_like(out_ref)

    h1 = _edge_h1(g_ref[...], enc_ref[...], w_ref, b1_ref, vt, et)
    h = h1 * a1_ref[...] + c1_ref[...]
    h = h * jax.nn.sigmoid(h)
    h2 = jnp.dot(h, w2_ref[...], preferred_element_type=jnp.float32) + b2_ref[...]
    out_ref[0:1, :] += jnp.sum(h2, axis=0, keepdims=True)
    out_ref[1:2, :] += jnp.sum(h2 * h2, axis=0, keepdims=True)


def _e3_body(g_ref, enc_ref, w_ref, b1_ref, a1_ref, c1_ref,
             w2_ref, b2_ref, a2_ref, c2_ref, out_ref, *, vt, et):
    h1 = _edge_h1(g_ref[...], enc_ref[...], w_ref, b1_ref, vt, et)
    h = h1 * a1_ref[...] + c1_ref[...]
    h = h * jax.nn.sigmoid(h)
    h2 = jnp.dot(h, w2_ref[...], preferred_element_type=jnp.float32) + b2_ref[...]
    y = h2 * a2_ref[...] + c2_ref[...]
    f2 = y[:, :H]
    c2 = y[:, H:]
    glu = jax.nn.sigmoid(f2) * jax.nn.softplus(c2)
    out_ref[...] = jnp.sum(glu.reshape(vt, K, H), axis=1)


def _knn_body(vp_ref, nvp_ref, ppt_ref, idx_ref, d_ref, a_ref, *, rt, npad):
    v = vp_ref[0]
    nv = nvp_ref[0]
    pt = ppt_ref[0]
    v4 = jnp.concatenate([-2.0 * v, jnp.ones((rt, 1), jnp.float32)], axis=1)
    s = jnp.dot(v4, pt, preferred_element_type=jnp.float32,
                precision=jax.lax.Precision.HIGHEST)
    q = jnp.dot(nv, pt[0:3, :], preferred_element_type=jnp.float32,
                precision=jax.lax.Precision.HIGHEST)
    vn2 = jnp.sum(v * v, axis=1, keepdims=True)
    vdn = jnp.sum(v * nv, axis=1, keepdims=True)
    iota = jax.lax.broadcasted_iota(jnp.int32, (rt, npad), 1)
    big = jnp.int32(2 ** 30)
    bigf = jnp.float32(3e38)
    icols, dcols, acols = [], [], []
    for _ in range(K):
        m = jnp.min(s, axis=1, keepdims=True)
        hit = s <= m
        icols.append(jnp.min(jnp.where(hit, iota, big), axis=1, keepdims=True))
        qs = jnp.min(jnp.where(hit, q, bigf), axis=1, keepdims=True)
        d = jnp.sqrt(jnp.maximum(m + vn2, 0.0))
        dcols.append(d)
        acols.append((qs - vdn) / d)
        s = jnp.where(hit, jnp.inf, s)
    idx_ref[0] = jnp.concatenate(icols, axis=1)
    d_ref[0] = jnp.concatenate(dcols, axis=1)
    a_ref[0] = jnp.concatenate(acols, axis=1)


def _knn(verts, node_pos, vnormals, n):
    npad = -(-n // 128) * 128
    rt = 128
    vp = jnp.pad(verts.reshape(B, n, 3), ((0, 0), (0, npad - n), (0, 0)))
    nvp = jnp.pad(vnormals.reshape(B, n, 3), ((0, 0), (0, npad - n), (0, 0)))
    pp = jnp.pad(node_pos.reshape(B, n, 3), ((0, 0), (0, npad - n), (0, 0)),
                 constant_values=1e6)
    ppt = jnp.transpose(pp, (0, 2, 1))
    ppt = jnp.concatenate(
        [ppt, jnp.sum(ppt * ppt, axis=1, keepdims=True)], axis=1)
    idx, d, a = pl.pallas_call(
        functools.partial(_knn_body, rt=rt, npad=npad),
        grid=(B, npad // rt),
        in_specs=[
            pl.BlockSpec((1, rt, 3), lambda b, r: (b, r, 0)),
            pl.BlockSpec((1, rt, 3), lambda b, r: (b, r, 0)),
            pl.BlockSpec((1, 4, npad), lambda b, r: (b, 0, 0)),
        ],
        out_specs=[
            pl.BlockSpec((1, rt, K), lambda b, r: (b, r, 0)),
            pl.BlockSpec((1, rt, K), lambda b, r: (b, r, 0)),
            pl.BlockSpec((1, rt, K), lambda b, r: (b, r, 0)),
        ],
        out_shape=[
            jax.ShapeDtypeStruct((B, npad, K), jnp.int32),
            jax.ShapeDtypeStruct((B, npad, K), jnp.float32),
            jax.ShapeDtypeStruct((B, npad, K), jnp.float32),
        ],
    )(vp, nvp, ppt)
    idx = idx[:, :n, :]
    idxf = (idx + (jnp.arange(B, dtype=jnp.int32) * n)[:, None, None]).reshape(-1)
    e = idxf.shape[0]
    dist_e = d[:, :n, :].reshape(e, 1)
    ang_e = a[:, :n, :].reshape(e, 1)
    geo = jnp.concatenate(
        [dist_e, ang_e, jnp.zeros((e, 14), jnp.float32)], axis=1)
    return idxf, geo


_NW = 32
_GC = 128


def _sc_gather(table, idxf):
    e = idxf.shape[0]
    td = table.shape[1]
    nchunks = e // _GC
    trips = -(-nchunks // _NW)
    mesh = plsc.VectorSubcoreMesh(core_axis_name="c", subcore_axis_name="s")

    def body(t_hbm, idx_hbm, out_hbm, idx_v, rows_v, sem):
        wid = lax.axis_index("c") * 16 + lax.axis_index("s")

        def step(i, _):
            g = wid + i * _NW

            @pl.when(g < nchunks)
            def _():
                base = g * _GC
                pltpu.sync_copy(idx_hbm.at[pl.ds(base, _GC)], idx_v)
                pltpu.async_copy(t_hbm.at[idx_v], rows_v, sem).wait()
                pltpu.sync_copy(rows_v, out_hbm.at[pl.ds(base, _GC)])
            return 0

        lax.fori_loop(0, trips, step, 0)

    return pl.kernel(
        body,
        out_type=jax.ShapeDtypeStruct((e, td), jnp.float32),
        mesh=mesh,
        scratch_types=[
            pltpu.VMEM((_GC,), jnp.int32),
            pltpu.VMEM((_GC, td), jnp.float32),
            pltpu.SemaphoreType.DMA,
        ],
    )(table, idxf)


def _bn_coefs(s, sq, count, g, be):
    mean = s / count
    var = sq / count - mean * mean
    a = g / jnp.sqrt(var + EPS)
    c = be - mean * a
    return a[None, :], c[None, :]


def _pick_vt(n_verts):
    for vt in (400, 200, 80, 40, 8):
        if n_verts % vt == 0:
            return vt
    return 8


def _edge_stage(gath, geo, n_verts, sw1r, sb1, sg1, sbe1, sw2, sb2, sg2, sbe2):
    e = gath.shape[0]
    vt = _pick_vt(n_verts)
    et = vt * K
    grid = n_verts // vt

    enc = pl.pallas_call(
        _enc_body, grid=(e // et,),
        in_specs=[pl.BlockSpec((et, 16), lambda i: (i, 0))],
        out_specs=pl.BlockSpec((et, 2 * NGDF), lambda i: (i, 0)),
        out_shape=jax.ShapeDtypeStruct((e, 2 * NGDF), jnp.float32),
    )(geo)

    espec = [
        pl.BlockSpec((et, H), lambda i: (i, 0)),
        pl.BlockSpec((et, 2 * NGDF), lambda i: (i, 0)),
        pl.BlockSpec((2 * NGDF, H), lambda i: (0, 0)),
        pl.BlockSpec((1, H), lambda i: (0, 0)),
    ]
    acc1 = pl.pallas_call(
        functools.partial(_e1_body, vt=vt, et=et),
        grid=(grid,),
        in_specs=espec,
        out_specs=pl.BlockSpec((2, H), lambda i: (0, 0)),
        out_shape=jax.ShapeDtypeStruct((2, H), jnp.float32),
    )(gath, enc, sw1r, sb1[None, :])
    a1, c1 = _bn_coefs(acc1[0], acc1[1], e, sg1, sbe1)

    espec2 = espec + [
        pl.BlockSpec((1, H), lambda i: (0, 0)),
        pl.BlockSpec((1, H), lambda i: (0, 0)),
        pl.BlockSpec((H, 2 * H), lambda i: (0, 0)),
        pl.BlockSpec((1, 2 * H), lambda i: (0, 0)),
    ]
    acc2 = pl.pallas_call(
        functools.partial(_e2_body, vt=vt, et=et),
        grid=(grid,),
        in_specs=espec2,
        out_specs=pl.BlockSpec((2, 2 * H), lambda i: (0, 0)),
        out_shape=jax.ShapeDtypeStruct((2, 2 * H), jnp.float32),
    )(gath, enc, sw1r, sb1[None, :], a1, c1, sw2, sb2[None, :])
    a2, c2 = _bn_coefs(acc2[0], acc2[1], e, sg2, sbe2)

    espec3 = espec2 + [
        pl.BlockSpec((1, 2 * H), lambda i: (0, 0)),
        pl.BlockSpec((1, 2 * H), lambda i: (0, 0)),
    ]
    hcg = pl.pallas_call(
        functools.partial(_e3_body, vt=vt, et=et),
        grid=(grid,),
        in_specs=espec3,
        out_specs=pl.BlockSpec((vt, H), lambda i: (i, 0)),
        out_shape=jax.ShapeDtypeStruct((n_verts, H), jnp.float32),
    )(gath, enc, sw1r, sb1[None, :], a1, c1, sw2, sb2[None, :], a2, c2)
    return hcg


def _acc_stats(s_ref, y, first):
    @pl.when(first)
    def _():
        s_ref[...] = jnp.zeros_like(s_ref)

    s_ref[0:1, :] += jnp.sum(y, axis=0, keepdims=True)
    s_ref[1:2, :] += jnp.sum(y * y, axis=0, keepdims=True)


def _lin_body(x_ref, w_ref, b_ref, y_ref, s_ref):
    y = jnp.dot(x_ref[...], w_ref[...],
                preferred_element_type=jnp.float32) + b_ref[...]
    y_ref[...] = y
    _acc_stats(s_ref, y, pl.program_id(0) == 0)


def _lin2_body(y_ref, a_ref, c_ref, w_ref, b_ref, y2_ref, s_ref):
    h = y_ref[...] * a_ref[...] + c_ref[...]
    h = h * jax.nn.sigmoid(h)
    y2 = jnp.dot(h, w_ref[...],
                 preferred_element_type=jnp.float32) + b_ref[...]
    y2_ref[...] = y2
    _acc_stats(s_ref, y2, pl.program_id(0) == 0)


def _chem_body(y_ref, a_ref, c_ref, w_ref, hc_ref, z_ref):
    yb = y_ref[...] * a_ref[...] + c_ref[...]
    hc = jax.nn.sigmoid(yb[:, :H]) * jax.nn.softplus(yb[:, H:])
    hc_ref[...] = hc
    z_ref[...] = jnp.dot(hc, w_ref[...], preferred_element_type=jnp.float32)


def _f1_body(hcg_ref, yg_ref, ag_ref, cg_ref, w_ref, b_ref, y_ref, s_ref):
    hg = yg_ref[...] * ag_ref[...] + cg_ref[...]
    x = jnp.concatenate([hcg_ref[...], hg], axis=1)
    y = jnp.dot(x, w_ref[...], preferred_element_type=jnp.float32) + b_ref[...]
    y_ref[...] = y
    _acc_stats(s_ref, y, pl.program_id(0) == 0)


def _bn_apply_body(y_ref, a_ref, c_ref, out_ref):
    out_ref[...] = y_ref[...] * a_ref[...] + c_ref[...]


def _pick_rows(n):
    for r in (2000, 400, 200, 80, 8):
        if n % r == 0:
            return r
    return 8


def _row_spec(rt, d):
    return pl.BlockSpec((rt, d), lambda i: (i, 0))


def _full_spec(r, d):
    return pl.BlockSpec((r, d), lambda i: (0, 0))


def _dense_call(body, n, rt, in_specs, outd, arrays, extra_outs=None):
    out_specs = [_row_spec(rt, outd), pl.BlockSpec((2, outd), lambda i: (0, 0))]
    out_shape = [jax.ShapeDtypeStruct((n, outd), jnp.float32),
                 jax.ShapeDtypeStruct((2, outd), jnp.float32)]
    return pl.pallas_call(
        body, grid=(n // rt,), in_specs=in_specs,
        out_specs=out_specs, out_shape=out_shape)(*arrays)


def _lin(x, w, b):
    n, ind = x.shape
    outd = w.shape[1]
    rt = _pick_rows(n)
    return _dense_call(
        _lin_body, n, rt,
        [_row_spec(rt, ind), _full_spec(ind, outd), _full_spec(1, outd)],
        outd, (x, w, b[None, :]))


def _lin2(y, a, c, w, b):
    n, ind = y.shape
    outd = w.shape[1]
    rt = _pick_rows(n)
    return _dense_call(
        _lin2_body, n, rt,
        [_row_spec(rt, ind), _full_spec(1, ind), _full_spec(1, ind),
         _full_spec(ind, outd), _full_spec(1, outd)],
        outd, (y, a, c, w, b[None, :]))


def _chem_apply(y, a, c, w):
    n = y.shape[0]
    rt = _pick_rows(n)
    return pl.pallas_call(
        _chem_body, grid=(n // rt,),
        in_specs=[_row_spec(rt, 2 * H), _full_spec(1, 2 * H),
                  _full_spec(1, 2 * H), _full_spec(H, H)],
        out_specs=[_row_spec(rt, H), _row_spec(rt, H)],
        out_shape=[jax.ShapeDtypeStruct((n, H), jnp.float32),
                   jax.ShapeDtypeStruct((n, H), jnp.float32)],
    )(y, a, c, w)


def _f1(hcg, yg, ag, cg, w, b):
    n = hcg.shape[0]
    rt = _pick_rows(n)
    return _dense_call(
        _f1_body, n, rt,
        [_row_spec(rt, H), _row_spec(rt, H), _full_spec(1, H),
         _full_spec(1, H), _full_spec(2 * H, H), _full_spec(1, H)],
        H, (hcg, yg, ag, cg, w, b[None, :]))


def _bn_apply(y, a, c):
    n, d = y.shape
    rt = _pick_rows(n)
    return pl.pallas_call(
        _bn_apply_body, grid=(n // rt,),
        in_specs=[_row_spec(rt, d), _full_spec(1, d), _full_spec(1, d)],
        out_specs=_row_spec(rt, d),
        out_shape=jax.ShapeDtypeStruct((n, d), jnp.float32),
    )(y, a, c)


def kernel(graph_x, surface_x, verts, node_pos, vnormals,
           cw1, cb1, cg1, cbe1, cw2, cb2, cg2, cbe2,
           gw1, gb1, gg1, gbe1, gw2, gb2, gg2, gbe2,
           sw1, sb1, sg1, sbe1, sw2, sb2, sg2, sbe2,
           fw1, fb1, fg1, fbe1, fw2, fb2, fg2, fbe2):
    n_total = verts.shape[0]
    n = n_total // B

    y_c1, st = _lin(graph_x, cw1, cb1)
    a, c = _bn_coefs(st[0], st[1], n_total, cg1, cbe1)
    y_c2, st = _lin2(y_c1, a, c, cw2, cb2)
    a, c = _bn_coefs(st[0], st[1], n_total, cg2, cbe2)
    h_chem, z = _chem_apply(y_c2, a, c, sw1[:H])

    y_g1, st = _lin(surface_x, gw1, gb1)
    a, c = _bn_coefs(st[0], st[1], n_total, gg1, gbe1)
    y_g2, st = _lin2(y_g1, a, c, gw2, gb2)
    ag2, cg2_ = _bn_coefs(st[0], st[1], n_total, gg2, gbe2)

    idxf, geo = _knn(verts, node_pos, vnormals, n)

    gath = _sc_gather(z, idxf)
    hcg = _edge_stage(gath, geo, n_total, sw1[H:], sb1, sg1, sbe1,
                      sw2, sb2, sg2, sbe2)

    y_f1, st = _f1(hcg, y_g2, ag2, cg2_, fw1, fb1)
    a, c = _bn_coefs(st[0], st[1], n_total, fg1, fbe1)
    y_f2, st = _lin2(y_f1, a, c, fw2, fb2)
    a, c = _bn_coefs(st[0], st[1], n_total, fg2, fbe2)
    h_geom = _bn_apply(y_f2, a, c)
    return (h_geom, h_chem)
